# baseline (speedup 1.0000x reference)
import functools

import jax
import jax.numpy as jnp
from jax import lax
from jax.experimental import pallas as pl
from jax.experimental.pallas import tpu as pltpu
from jax.experimental.pallas import tpu_sc as plsc

_B, _N, _K = 4, 10000, 16
_GROUPS = 8
_PB = 400


def _leaky(x):
    return jnp.where(x >= 0, x, 0.1 * x)


def _mm(w, x):
    return lax.dot_general(w, x, (((1,), (0,)), ((), ())))


def _stage_a(xyz_ref, feat_ref, w1x_ref, w1f_ref, b1_ref, h1_ref, p1_ref):
    p1 = _mm(w1x_ref[...], xyz_ref[0])
    h1_ref[0] = p1 + _mm(w1f_ref[...], feat_ref[0])
    p1_ref[0] = p1 - b1_ref[...]


def _stage_b(m1_ref, p1_ref, xyz_ref, w2x_ref, w2f_ref, b2_ref, h2_ref, p2_ref):
    f1 = _leaky(m1_ref[0] - p1_ref[0])
    p2 = _mm(w2x_ref[...], xyz_ref[0])
    h2_ref[0] = p2 + _mm(w2f_ref[...], f1)
    p2_ref[0] = p2 - b2_ref[...]


def _stage_c(m2_ref, p2_ref, wm1_ref, bm1_ref, wm2_ref, bm2_ref, wl_ref,
             bl_ref, h_ref, flow_ref):
    f2 = _leaky(m2_ref[0] - p2_ref[0])
    h1 = _leaky(_mm(wm1_ref[...], f2) + bm1_ref[...])
    h = _leaky(_mm(wm2_ref[...], h1) + bm2_ref[...])
    h_ref[0] = h
    flow_ref[0] = _mm(wl_ref[...], h) + bl_ref[...]


def _batch_spec(c, n):
    return pl.BlockSpec((1, c, n), lambda b: (b, 0, 0))


def _full_spec(*shape):
    return pl.BlockSpec(shape, lambda b: tuple(0 for _ in shape))


def _run_stage_a(xyz8, feat, w1x, w1f, b1c):
    return pl.pallas_call(
        _stage_a,
        grid=(_B,),
        in_specs=[_batch_spec(8, _N), _batch_spec(128, _N),
                  _full_spec(64, 8), _full_spec(64, 128), _full_spec(64, 1)],
        out_specs=[_batch_spec(64, _N), _batch_spec(64, _N)],
        out_shape=[jax.ShapeDtypeStruct((_B, 64, _N), jnp.float32),
                   jax.ShapeDtypeStruct((_B, 64, _N), jnp.float32)],
    )(xyz8, feat, w1x, w1f, b1c)


def _run_stage_b(m1, p1, xyz8, w2x, w2f, b2c):
    return pl.pallas_call(
        _stage_b,
        grid=(_B,),
        in_specs=[_batch_spec(64, _N), _batch_spec(64, _N), _batch_spec(8, _N),
                  _full_spec(64, 8), _full_spec(64, 64), _full_spec(64, 1)],
        out_specs=[_batch_spec(64, _N), _batch_spec(64, _N)],
        out_shape=[jax.ShapeDtypeStruct((_B, 64, _N), jnp.float32),
                   jax.ShapeDtypeStruct((_B, 64, _N), jnp.float32)],
    )(m1, p1, xyz8, w2x, w2f, b2c)


def _run_stage_c(m2, p2, wm1, bm1c, wm2, bm2c, wl, blc):
    return pl.pallas_call(
        _stage_c,
        grid=(_B,),
        in_specs=[_batch_spec(64, _N), _batch_spec(64, _N),
                  _full_spec(64, 64), _full_spec(64, 1),
                  _full_spec(64, 64), _full_spec(64, 1),
                  _full_spec(3, 64), _full_spec(3, 1)],
        out_specs=[_batch_spec(64, _N), _batch_spec(3, _N)],
        out_shape=[jax.ShapeDtypeStruct((_B, 64, _N), jnp.float32),
                   jax.ShapeDtypeStruct((_B, 3, _N), jnp.float32)],
    )(m2, p2, wm1, bm1c, wm2, bm2c, wl, blc)


@functools.cache
def _build_gather_max():
    mesh = plsc.VectorSubcoreMesh(core_axis_name="c", subcore_axis_name="s")
    return functools.partial(
        pl.kernel,
        mesh=mesh,
        compiler_params=pltpu.CompilerParams(
            use_tc_tiling_on_sc=False, needs_layout_passes=False),
        out_type=jax.ShapeDtypeStruct((_B * 64 * _N,), jnp.float32),
        scratch_types=[
            pltpu.VMEM((8 * _N,), jnp.float32),
            pltpu.VMEM((_PB * _K,), jnp.int32),
            pltpu.VMEM((8 * _PB,), jnp.float32),
        ],
    )(_gather_max_body)


def _gather_max(h, knn):
    m = _build_gather_max()(h.reshape(-1), knn.reshape(-1))
    return m.reshape(_B, 64, _N)


def _gather_max_body(h_hbm, knn_hbm, m_hbm, table_v, knn_v, out_v):
    cid = lax.axis_index("c")
    sid = lax.axis_index("s")
    wid = sid * 2 + cid
    b = wid // _GROUPS
    g = wid % _GROUPS
    row0 = (b * 64 + g * 8) * _N

    pltpu.sync_copy(h_hbm.at[pl.ds(row0, 8 * _N)], table_v)

    iota = lax.iota(jnp.int32, 16)

    def block_body(blk, carry):
        n0 = blk * _PB
        pltpu.sync_copy(knn_hbm.at[pl.ds((b * _N + n0) * _K, _PB * _K)],
                        knn_v)

        def sub(j, carry2):
            p0 = j * 16
            rows16 = (iota + p0) * _K
            cols = [plsc.load_gather(knn_v, [rows16 + k]) for k in range(_K)]
            for c in range(8):
                acc = plsc.load_gather(table_v, [cols[0] + c * _N])
                for k in range(1, _K):
                    acc = jnp.maximum(
                        acc, plsc.load_gather(table_v, [cols[k] + c * _N]))
                out_v[pl.ds(c * _PB + p0, 16)] = acc
            return carry2

        lax.fori_loop(0, _PB // 16, sub, 0)
        for c in range(8):
            pltpu.sync_copy(out_v.at[pl.ds(c * _PB, _PB)],
                            m_hbm.at[pl.ds(row0 + c * _N + n0, _PB)])
        return carry

    lax.fori_loop(0, _N // _PB, block_body, 0)


def kernel(xyz, feat, knn_indices, mask, W1, b1, W2, b2, Wm1, bm1, Wm2, bm2,
           Wl, bl):
    del mask
    knn = knn_indices.astype(jnp.int32)
    xyz8 = jnp.pad(xyz, ((0, 0), (0, 5), (0, 0)))
    w1x = jnp.pad(W1[:, :3], ((0, 0), (0, 5)))
    w2x = jnp.pad(W2[:, :3], ((0, 0), (0, 5)))

    h1, p1 = _run_stage_a(xyz8, feat, w1x, W1[:, 3:], b1[:, None])
    m1 = _gather_max(h1, knn)
    h2, p2 = _run_stage_b(m1, p1, xyz8, w2x, W2[:, 3:], b2[:, None])
    m2 = _gather_max(h2, knn)
    h, flow = _run_stage_c(m2, p2, Wm1, bm1[:, None], Wm2, bm2[:, None],
                           Wl, bl[:, None])
    return (h, flow)


# speedup vs baseline: 1.2373x; 1.2373x over previous
import functools

import jax
import jax.numpy as jnp
from jax import lax
from jax.experimental import pallas as pl
from jax.experimental.pallas import tpu as pltpu
from jax.experimental.pallas import tpu_sc as plsc

_B, _N, _K = 4, 10000, 16
_GROUPS = 8
_PB = 400
_NB = _N // _PB


def _leaky(x):
    return jnp.where(x >= 0, x, 0.1 * x)


def _mm(w, x):
    return lax.dot_general(w, x, (((1,), (0,)), ((), ())))


def _stage_a(xyz_ref, feat_ref, w1x_ref, w1f_ref, h1_ref):
    h1_ref[0] = _mm(w1x_ref[...], xyz_ref[0]) + _mm(w1f_ref[...], feat_ref[0])


def _stage_b(m1_ref, xyz_ref, w1x_ref, b1_ref, w2x_ref, w2f_ref, h2_ref):
    f1 = _leaky(m1_ref[0] - _mm(w1x_ref[...], xyz_ref[0]) + b1_ref[...])
    h2_ref[0] = _mm(w2x_ref[...], xyz_ref[0]) + _mm(w2f_ref[...], f1)


def _stage_c(m2_ref, xyz_ref, w2x_ref, b2_ref, wm1_ref, bm1_ref, wm2_ref,
             bm2_ref, wl_ref, bl_ref, h_ref, flow_ref):
    f2 = _leaky(m2_ref[0] - _mm(w2x_ref[...], xyz_ref[0]) + b2_ref[...])
    h1 = _leaky(_mm(wm1_ref[...], f2) + bm1_ref[...])
    h = _leaky(_mm(wm2_ref[...], h1) + bm2_ref[...])
    h_ref[0] = h
    flow_ref[0] = _mm(wl_ref[...], h) + bl_ref[...]


def _batch_spec(c, n):
    return pl.BlockSpec((1, c, n), lambda b: (b, 0, 0))


def _full_spec(*shape):
    return pl.BlockSpec(shape, lambda b: tuple(0 for _ in shape))


def _run_stage_a(xyz8, feat, w1x, w1f):
    return pl.pallas_call(
        _stage_a,
        grid=(_B,),
        in_specs=[_batch_spec(8, _N), _batch_spec(128, _N),
                  _full_spec(64, 8), _full_spec(64, 128)],
        out_specs=_batch_spec(64, _N),
        out_shape=jax.ShapeDtypeStruct((_B, 64, _N), jnp.float32),
    )(xyz8, feat, w1x, w1f)


def _run_stage_b(m1, xyz8, w1x, b1c, w2x, w2f):
    return pl.pallas_call(
        _stage_b,
        grid=(_B,),
        in_specs=[_batch_spec(64, _N), _batch_spec(8, _N),
                  _full_spec(64, 8), _full_spec(64, 1),
                  _full_spec(64, 8), _full_spec(64, 64)],
        out_specs=_batch_spec(64, _N),
        out_shape=jax.ShapeDtypeStruct((_B, 64, _N), jnp.float32),
    )(m1, xyz8, w1x, b1c, w2x, w2f)


def _run_stage_c(m2, xyz8, w2x, b2c, wm1, bm1c, wm2, bm2c, wl, blc):
    return pl.pallas_call(
        _stage_c,
        grid=(_B,),
        in_specs=[_batch_spec(64, _N), _batch_spec(8, _N),
                  _full_spec(64, 8), _full_spec(64, 1),
                  _full_spec(64, 64), _full_spec(64, 1),
                  _full_spec(64, 64), _full_spec(64, 1),
                  _full_spec(3, 64), _full_spec(3, 1)],
        out_specs=[_batch_spec(64, _N), _batch_spec(3, _N)],
        out_shape=[jax.ShapeDtypeStruct((_B, 64, _N), jnp.float32),
                   jax.ShapeDtypeStruct((_B, 3, _N), jnp.float32)],
    )(m2, xyz8, w2x, b2c, wm1, bm1c, wm2, bm2c, wl, blc)


@functools.cache
def _build_gather_max():
    mesh = plsc.VectorSubcoreMesh(core_axis_name="c", subcore_axis_name="s")
    return functools.partial(
        pl.kernel,
        mesh=mesh,
        compiler_params=pltpu.CompilerParams(
            use_tc_tiling_on_sc=False, needs_layout_passes=False),
        out_type=jax.ShapeDtypeStruct((_B * 64, _N), jnp.float32),
        scratch_types=[
            pltpu.VMEM((8, _N), jnp.float32),
            pltpu.VMEM((2, _PB, _K), jnp.int32),
            pltpu.VMEM((2, 8, _PB), jnp.float32),
            pltpu.SemaphoreType.DMA,
            pltpu.SemaphoreType.DMA,
            pltpu.SemaphoreType.DMA,
            pltpu.SemaphoreType.DMA,
        ],
    )(_gather_max_body)


def _gather_max(h, knn):
    m = _build_gather_max()(h.reshape(_B * 64, _N), knn)
    return m.reshape(_B, 64, _N)


def _gather_max_body(h_hbm, knn_hbm, m_hbm, table_v, knn_v, out_v,
                     sem_k0, sem_k1, sem_o0, sem_o1):
    cid = lax.axis_index("c")
    sid = lax.axis_index("s")
    wid = sid * 2 + cid
    b = wid // _GROUPS
    g = wid % _GROUPS
    row0 = b * 64 + g * 8

    pltpu.sync_copy(h_hbm.at[pl.ds(row0, 8), :], table_v)

    sem_k = (sem_k0, sem_k1)
    sem_o = (sem_o0, sem_o1)
    iota = lax.iota(jnp.int32, 16)
    ksplats = [jnp.full((16,), k, jnp.int32) for k in range(_K)]
    csplats = [jnp.full((16,), c, jnp.int32) for c in range(8)]

    def _knn_copy(blk, s):
        return pltpu.make_async_copy(
            knn_hbm.at[b, pl.ds(blk * _PB, _PB), :], knn_v.at[s], sem_k[s])

    def _out_copy(blk, s):
        return pltpu.make_async_copy(
            out_v.at[s], m_hbm.at[pl.ds(row0, 8), pl.ds(blk * _PB, _PB)],
            sem_o[s])

    _knn_copy(0, 0).start()

    def outer(gi, carry):
        for s in range(2):
            blk = gi * 2 + s

            @pl.when(blk < _NB)
            def _process():
                @pl.when(blk + 1 < _NB)
                def _prefetch():
                    _knn_copy(blk + 1, 1 - s).start()

                _knn_copy(blk, s).wait()

                @pl.when(blk >= 2)
                def _drain():
                    _out_copy(blk - 2, s).wait()

                ssplat = jnp.full((16,), s, jnp.int32)

                def sub(j, carry2):
                    p0 = j * 16
                    rows = iota + p0
                    cols = [plsc.load_gather(knn_v, [ssplat, rows, ksplats[k]])
                            for k in range(_K)]
                    for c in range(8):
                        acc = plsc.load_gather(table_v, [csplats[c], cols[0]])
                        for k in range(1, _K):
                            acc = jnp.maximum(
                                acc,
                                plsc.load_gather(table_v,
                                                 [csplats[c], cols[k]]))
                        out_v[s, c, pl.ds(p0, 16)] = acc
                    return carry2

                lax.fori_loop(0, _PB // 16, sub, 0)
                _out_copy(blk, s).start()
        return carry

    lax.fori_loop(0, (_NB + 1) // 2, outer, 0)
    _out_copy(_NB - 2, (_NB - 2) % 2).wait()
    _out_copy(_NB - 1, (_NB - 1) % 2).wait()


def kernel(xyz, feat, knn_indices, mask, W1, b1, W2, b2, Wm1, bm1, Wm2, bm2,
           Wl, bl):
    del mask
    knn = knn_indices.astype(jnp.int32)
    xyz8 = jnp.pad(xyz, ((0, 0), (0, 5), (0, 0)))
    w1x = jnp.pad(W1[:, :3], ((0, 0), (0, 5)))
    w2x = jnp.pad(W2[:, :3], ((0, 0), (0, 5)))

    h1 = _run_stage_a(xyz8, feat, w1x, W1[:, 3:])
    m1 = _gather_max(h1, knn)
    h2 = _run_stage_b(m1, xyz8, w1x, b1[:, None], w2x, W2[:, 3:])
    m2 = _gather_max(h2, knn)
    h, flow = _run_stage_c(m2, xyz8, w2x, b2[:, None], Wm1, bm1[:, None],
                           Wm2, bm2[:, None], Wl, bl[:, None])
    return (h, flow)


# speedup vs baseline: 1.4454x; 1.1681x over previous
import functools

import jax
import jax.numpy as jnp
from jax import lax
from jax.experimental import pallas as pl
from jax.experimental.pallas import tpu as pltpu
from jax.experimental.pallas import tpu_sc as plsc

_B, _N, _K = 4, 10000, 16
_GROUPS = 8
_PB = 400
_NB = _N // _PB


def _leaky(x):
    return jnp.where(x >= 0, x, 0.1 * x)


def _mm(w, x):
    return lax.dot_general(w, x, (((1,), (0,)), ((), ())))


def _stage_a(xyz_ref, feat_ref, w1x_ref, w1f_ref, h1_ref):
    h1_ref[0] = _mm(w1x_ref[...], xyz_ref[0]) + _mm(w1f_ref[...], feat_ref[0])


def _stage_b(m1_ref, xyz_ref, w1x_ref, b1_ref, w2x_ref, w2f_ref, h2_ref):
    f1 = _leaky(m1_ref[0] - _mm(w1x_ref[...], xyz_ref[0]) + b1_ref[...])
    h2_ref[0] = _mm(w2x_ref[...], xyz_ref[0]) + _mm(w2f_ref[...], f1)


def _stage_c(m2_ref, xyz_ref, w2x_ref, b2_ref, wm1_ref, bm1_ref, wm2_ref,
             bm2_ref, wl_ref, bl_ref, h_ref, flow_ref):
    f2 = _leaky(m2_ref[0] - _mm(w2x_ref[...], xyz_ref[0]) + b2_ref[...])
    h1 = _leaky(_mm(wm1_ref[...], f2) + bm1_ref[...])
    h = _leaky(_mm(wm2_ref[...], h1) + bm2_ref[...])
    h_ref[0] = h
    flow_ref[0] = _mm(wl_ref[...], h) + bl_ref[...]


def _batch_spec(c, n):
    return pl.BlockSpec((1, c, n), lambda b: (b, 0, 0))


def _full_spec(*shape):
    return pl.BlockSpec(shape, lambda b: tuple(0 for _ in shape))


def _run_stage_a(xyz8, feat, w1x, w1f):
    return pl.pallas_call(
        _stage_a,
        grid=(_B,),
        in_specs=[_batch_spec(8, _N), _batch_spec(128, _N),
                  _full_spec(64, 8), _full_spec(64, 128)],
        out_specs=_batch_spec(64, _N),
        out_shape=jax.ShapeDtypeStruct((_B, 64, _N), jnp.float32),
    )(xyz8, feat, w1x, w1f)


def _run_stage_b(m1, xyz8, w1x, b1c, w2x, w2f):
    return pl.pallas_call(
        _stage_b,
        grid=(_B,),
        in_specs=[_batch_spec(64, _N), _batch_spec(8, _N),
                  _full_spec(64, 8), _full_spec(64, 1),
                  _full_spec(64, 8), _full_spec(64, 64)],
        out_specs=_batch_spec(64, _N),
        out_shape=jax.ShapeDtypeStruct((_B, 64, _N), jnp.float32),
    )(m1, xyz8, w1x, b1c, w2x, w2f)


def _run_stage_c(m2, xyz8, w2x, b2c, wm1, bm1c, wm2, bm2c, wl, blc):
    return pl.pallas_call(
        _stage_c,
        grid=(_B,),
        in_specs=[_batch_spec(64, _N), _batch_spec(8, _N),
                  _full_spec(64, 8), _full_spec(64, 1),
                  _full_spec(64, 64), _full_spec(64, 1),
                  _full_spec(64, 64), _full_spec(64, 1),
                  _full_spec(3, 64), _full_spec(3, 1)],
        out_specs=[_batch_spec(64, _N), _batch_spec(3, _N)],
        out_shape=[jax.ShapeDtypeStruct((_B, 64, _N), jnp.float32),
                   jax.ShapeDtypeStruct((_B, 3, _N), jnp.float32)],
    )(m2, xyz8, w2x, b2c, wm1, bm1c, wm2, bm2c, wl, blc)


@functools.cache
def _build_gather_max():
    mesh = plsc.VectorSubcoreMesh(core_axis_name="c", subcore_axis_name="s")
    return functools.partial(
        pl.kernel,
        mesh=mesh,
        compiler_params=pltpu.CompilerParams(
            use_tc_tiling_on_sc=False, needs_layout_passes=False),
        out_type=jax.ShapeDtypeStruct((_B * 64, _N), jnp.float32),
        scratch_types=[
            pltpu.VMEM((8, _N), jnp.float32),
            pltpu.VMEM((2, _K, _PB), jnp.int32),
            pltpu.VMEM((2, 8, _PB), jnp.float32),
            pltpu.SemaphoreType.DMA,
            pltpu.SemaphoreType.DMA,
            pltpu.SemaphoreType.DMA,
            pltpu.SemaphoreType.DMA,
        ],
    )(_gather_max_body)


def _gather_max(h, knn_t):
    m = _build_gather_max()(h.reshape(_B * 64, _N), knn_t)
    return m.reshape(_B, 64, _N)


def _gather_max_body(h_hbm, knn_hbm, m_hbm, table_v, knn_v, out_v,
                     sem_k0, sem_k1, sem_o0, sem_o1):
    cid = lax.axis_index("c")
    sid = lax.axis_index("s")
    wid = sid * 2 + cid
    b = wid // _GROUPS
    g = wid % _GROUPS
    row0 = b * 64 + g * 8

    pltpu.sync_copy(h_hbm.at[pl.ds(row0, 8), :], table_v)

    sem_k = (sem_k0, sem_k1)
    sem_o = (sem_o0, sem_o1)
    csplats = [jnp.full((16,), c, jnp.int32) for c in range(8)]

    def _knn_copy(blk, s):
        return pltpu.make_async_copy(
            knn_hbm.at[b, :, pl.ds(blk * _PB, _PB)], knn_v.at[s], sem_k[s])

    def _out_copy(blk, s):
        return pltpu.make_async_copy(
            out_v.at[s], m_hbm.at[pl.ds(row0, 8), pl.ds(blk * _PB, _PB)],
            sem_o[s])

    _knn_copy(0, 0).start()

    def outer(gi, carry):
        for s in range(2):
            blk = gi * 2 + s

            @pl.when(blk < _NB)
            def _process():
                @pl.when(blk + 1 < _NB)
                def _prefetch():
                    _knn_copy(blk + 1, 1 - s).start()

                _knn_copy(blk, s).wait()

                @pl.when(blk >= 2)
                def _drain():
                    _out_copy(blk - 2, s).wait()

                def sub(j, carry2):
                    p0 = j * 16
                    cols = [knn_v[s, k, pl.ds(p0, 16)] for k in range(_K)]
                    for c in range(8):
                        acc = plsc.load_gather(table_v, [csplats[c], cols[0]])
                        for k in range(1, _K):
                            acc = jnp.maximum(
                                acc,
                                plsc.load_gather(table_v,
                                                 [csplats[c], cols[k]]))
                        out_v[s, c, pl.ds(p0, 16)] = acc
                    return carry2

                lax.fori_loop(0, _PB // 16, sub, 0)
                _out_copy(blk, s).start()
        return carry

    lax.fori_loop(0, (_NB + 1) // 2, outer, 0)
    _out_copy(_NB - 2, (_NB - 2) % 2).wait()
    _out_copy(_NB - 1, (_NB - 1) % 2).wait()


def kernel(xyz, feat, knn_indices, mask, W1, b1, W2, b2, Wm1, bm1, Wm2, bm2,
           Wl, bl):
    del mask
    knn = knn_indices.astype(jnp.int32)
    xyz8 = jnp.pad(xyz, ((0, 0), (0, 5), (0, 0)))
    w1x = jnp.pad(W1[:, :3], ((0, 0), (0, 5)))
    w2x = jnp.pad(W2[:, :3], ((0, 0), (0, 5)))

    knn_t = knn.transpose(0, 2, 1)
    h1 = _run_stage_a(xyz8, feat, w1x, W1[:, 3:])
    m1 = _gather_max(h1, knn_t)
    h2 = _run_stage_b(m1, xyz8, w1x, b1[:, None], w2x, W2[:, 3:])
    m2 = _gather_max(h2, knn_t)
    h, flow = _run_stage_c(m2, xyz8, w2x, b2[:, None], Wm1, bm1[:, None],
                           Wm2, bm2[:, None], Wl, bl[:, None])
    return (h, flow)


# speedup vs baseline: 2.2730x; 1.5726x over previous
import functools

import jax
import jax.numpy as jnp
from jax import lax
from jax.experimental import pallas as pl
from jax.experimental.pallas import tpu as pltpu
from jax.experimental.pallas import tpu_sc as plsc

_B, _N, _K = 4, 10000, 16
_GROUPS = 8
_PB = 400
_NB = _N // _PB


def _leaky(x):
    return jnp.where(x >= 0, x, 0.1 * x)


def _mm(w, x):
    return lax.dot_general(w, x, (((1,), (0,)), ((), ())))


_HI_MASK = 0xFFFF0000


def _pack(h):
    hb = lax.convert_element_type(
        lax.convert_element_type(h, jnp.bfloat16), jnp.float32)
    u = lax.bitcast_convert_type(hb, jnp.uint32)
    return lax.bitcast_convert_type(
        (u[:32] >> 16) | (u[32:] & jnp.uint32(_HI_MASK)), jnp.int32)


def _unpack(mp):
    u = lax.bitcast_convert_type(mp, jnp.uint32)
    lo = lax.bitcast_convert_type(u << 16, jnp.float32)
    hi = lax.bitcast_convert_type(u & jnp.uint32(_HI_MASK), jnp.float32)
    return lax.concatenate([lo, hi], 0)


def _stage_a(xyz_ref, feat_ref, w1x_ref, w1f_ref, h1_ref):
    h1 = _mm(w1x_ref[...], xyz_ref[0]) + _mm(w1f_ref[...], feat_ref[0])
    h1_ref[0] = _pack(h1)


def _stage_b(m1_ref, xyz_ref, w1x_ref, b1_ref, w2x_ref, w2f_ref, h2_ref):
    f1 = _leaky(_unpack(m1_ref[0]) - _mm(w1x_ref[...], xyz_ref[0])
                + b1_ref[...])
    h2_ref[0] = _pack(_mm(w2x_ref[...], xyz_ref[0]) + _mm(w2f_ref[...], f1))


def _stage_c(m2_ref, xyz_ref, w2x_ref, b2_ref, wm1_ref, bm1_ref, wm2_ref,
             bm2_ref, wl_ref, bl_ref, h_ref, flow_ref):
    f2 = _leaky(_unpack(m2_ref[0]) - _mm(w2x_ref[...], xyz_ref[0])
                + b2_ref[...])
    h1 = _leaky(_mm(wm1_ref[...], f2) + bm1_ref[...])
    h = _leaky(_mm(wm2_ref[...], h1) + bm2_ref[...])
    h_ref[0] = h
    flow_ref[0] = _mm(wl_ref[...], h) + bl_ref[...]


def _batch_spec(c, n):
    return pl.BlockSpec((1, c, n), lambda b: (b, 0, 0))


def _full_spec(*shape):
    return pl.BlockSpec(shape, lambda b: tuple(0 for _ in shape))


def _run_stage_a(xyz8, feat, w1x, w1f):
    return pl.pallas_call(
        _stage_a,
        grid=(_B,),
        in_specs=[_batch_spec(8, _N), _batch_spec(128, _N),
                  _full_spec(64, 8), _full_spec(64, 128)],
        out_specs=_batch_spec(32, _N),
        out_shape=jax.ShapeDtypeStruct((_B, 32, _N), jnp.int32),
    )(xyz8, feat, w1x, w1f)


def _run_stage_b(m1, xyz8, w1x, b1c, w2x, w2f):
    return pl.pallas_call(
        _stage_b,
        grid=(_B,),
        in_specs=[_batch_spec(32, _N), _batch_spec(8, _N),
                  _full_spec(64, 8), _full_spec(64, 1),
                  _full_spec(64, 8), _full_spec(64, 64)],
        out_specs=_batch_spec(32, _N),
        out_shape=jax.ShapeDtypeStruct((_B, 32, _N), jnp.int32),
    )(m1, xyz8, w1x, b1c, w2x, w2f)


def _run_stage_c(m2, xyz8, w2x, b2c, wm1, bm1c, wm2, bm2c, wl, blc):
    return pl.pallas_call(
        _stage_c,
        grid=(_B,),
        in_specs=[_batch_spec(32, _N), _batch_spec(8, _N),
                  _full_spec(64, 8), _full_spec(64, 1),
                  _full_spec(64, 64), _full_spec(64, 1),
                  _full_spec(64, 64), _full_spec(64, 1),
                  _full_spec(3, 64), _full_spec(3, 1)],
        out_specs=[_batch_spec(64, _N), _batch_spec(3, _N)],
        out_shape=[jax.ShapeDtypeStruct((_B, 64, _N), jnp.float32),
                   jax.ShapeDtypeStruct((_B, 3, _N), jnp.float32)],
    )(m2, xyz8, w2x, b2c, wm1, bm1c, wm2, bm2c, wl, blc)


@functools.cache
def _build_gather_max():
    mesh = plsc.VectorSubcoreMesh(core_axis_name="c", subcore_axis_name="s")
    return functools.partial(
        pl.kernel,
        mesh=mesh,
        compiler_params=pltpu.CompilerParams(
            use_tc_tiling_on_sc=False, needs_layout_passes=False),
        out_type=jax.ShapeDtypeStruct((_B * 32, _N), jnp.int32),
        scratch_types=[
            pltpu.VMEM((4, _N), jnp.int32),
            pltpu.VMEM((2, _K, _PB), jnp.int32),
            pltpu.VMEM((2, 4, _PB), jnp.int32),
            pltpu.SemaphoreType.DMA,
            pltpu.SemaphoreType.DMA,
            pltpu.SemaphoreType.DMA,
            pltpu.SemaphoreType.DMA,
        ],
    )(_gather_max_body)


def _gather_max(hp, knn_t):
    m = _build_gather_max()(hp.reshape(_B * 32, _N), knn_t)
    return m.reshape(_B, 32, _N)


def _gather_max_body(h_hbm, knn_hbm, m_hbm, table_v, knn_v, out_v,
                     sem_k0, sem_k1, sem_o0, sem_o1):
    cid = lax.axis_index("c")
    sid = lax.axis_index("s")
    wid = sid * 2 + cid
    b = wid // _GROUPS
    g = wid % _GROUPS
    row0 = b * 32 + g * 4

    pltpu.sync_copy(h_hbm.at[pl.ds(row0, 4), :], table_v)

    sem_k = (sem_k0, sem_k1)
    sem_o = (sem_o0, sem_o1)
    psplats = [jnp.full((16,), p, jnp.int32) for p in range(4)]

    def _knn_copy(blk, s):
        return pltpu.make_async_copy(
            knn_hbm.at[b, :, pl.ds(blk * _PB, _PB)], knn_v.at[s], sem_k[s])

    def _out_copy(blk, s):
        return pltpu.make_async_copy(
            out_v.at[s], m_hbm.at[pl.ds(row0, 4), pl.ds(blk * _PB, _PB)],
            sem_o[s])

    _knn_copy(0, 0).start()

    def outer(gi, carry):
        for s in range(2):
            blk = gi * 2 + s

            @pl.when(blk < _NB)
            def _process():
                @pl.when(blk + 1 < _NB)
                def _prefetch():
                    _knn_copy(blk + 1, 1 - s).start()

                _knn_copy(blk, s).wait()

                @pl.when(blk >= 2)
                def _drain():
                    _out_copy(blk - 2, s).wait()

                def sub(j, carry2):
                    p0 = j * 16
                    cols = [knn_v[s, k, pl.ds(p0, 16)] for k in range(_K)]
                    for p in range(4):
                        acc = plsc.bitcast(
                            plsc.load_gather(table_v, [psplats[p], cols[0]]),
                            jnp.bfloat16)
                        for k in range(1, _K):
                            acc = jnp.maximum(acc, plsc.bitcast(
                                plsc.load_gather(table_v,
                                                 [psplats[p], cols[k]]),
                                jnp.bfloat16))
                        out_v[s, p, pl.ds(p0, 16)] = plsc.bitcast(
                            acc, jnp.int32)
                    return carry2

                lax.fori_loop(0, _PB // 16, sub, 0)
                _out_copy(blk, s).start()
        return carry

    lax.fori_loop(0, (_NB + 1) // 2, outer, 0)
    _out_copy(_NB - 2, (_NB - 2) % 2).wait()
    _out_copy(_NB - 1, (_NB - 1) % 2).wait()


def kernel(xyz, feat, knn_indices, mask, W1, b1, W2, b2, Wm1, bm1, Wm2, bm2,
           Wl, bl):
    del mask
    knn = knn_indices.astype(jnp.int32)
    xyz8 = jnp.pad(xyz, ((0, 0), (0, 5), (0, 0)))
    w1x = jnp.pad(W1[:, :3], ((0, 0), (0, 5)))
    w2x = jnp.pad(W2[:, :3], ((0, 0), (0, 5)))

    knn_t = knn.transpose(0, 2, 1)
    h1 = _run_stage_a(xyz8, feat, w1x, W1[:, 3:])
    m1 = _gather_max(h1, knn_t)
    h2 = _run_stage_b(m1, xyz8, w1x, b1[:, None], w2x, W2[:, 3:])
    m2 = _gather_max(h2, knn_t)
    h, flow = _run_stage_c(m2, xyz8, w2x, b2[:, None], Wm1, bm1[:, None],
                           Wm2, bm2[:, None], Wl, bl[:, None])
    return (h, flow)
